# baseline (speedup 1.0000x reference)
import functools

import jax
import jax.numpy as jnp
from jax import lax
from jax.experimental import pallas as pl
from jax.experimental.pallas import tpu as pltpu
from jax.experimental.pallas import tpu_sc as plsc

_B = 4
_L = 8192
_EMB = 64
_NC = 2
_NS = 16
_NW = _NC * _NS
_ROWS = _B * _L
_CHUNK = _ROWS // _NW
_GCH = 128
_NG = _CHUNK // _GCH


def _sc_gather(idx2d, token_table):
    mesh = plsc.VectorSubcoreMesh(
        core_axis_name="c", subcore_axis_name="s", num_cores=1
    )

    @functools.partial(
        pl.kernel,
        mesh=mesh,
        out_type=jax.ShapeDtypeStruct((_ROWS, _EMB), jnp.float32),
        scratch_types=[
            pltpu.VMEM((16, _GCH), jnp.int32),
            pltpu.VMEM((512, _EMB), jnp.float32),
            pltpu.VMEM((512, _EMB), jnp.float32),
            pltpu.SemaphoreType.DMA,
            pltpu.SemaphoreType.DMA,
        ],
        compiler_params=pltpu.CompilerParams(use_tc_tiling_on_sc=False),
    )
    def k(idx_hbm, tok_hbm, out_hbm, idx_v, rows_a, rows_b, sem_a, sem_b):
        w = lax.axis_index("s")
        pltpu.sync_copy(idx_hbm.at[pl.ds(w * 16, 16)], idx_v)
        bufs = (rows_a, rows_b)
        sems = (sem_a, sem_b)

        def fire(q):
            buf, sem = bufs[q % 2], sems[q % 2]
            return [
                pltpu.async_copy(
                    tok_hbm.at[idx_v.at[q * 4 + h]],
                    buf.at[pl.ds(h * _GCH, _GCH)],
                    sem,
                )
                for h in range(4)
            ]

        def drain(q, cps):
            for cp in cps:
                cp.wait()
            pltpu.sync_copy(
                bufs[q % 2], out_hbm.at[pl.ds(w * 2048 + q * 512, 512)]
            )

        cp0 = fire(0)
        cp1 = fire(1)
        drain(0, cp0)
        cp2 = fire(2)
        drain(1, cp1)
        cp3 = fire(3)
        drain(2, cp2)
        drain(3, cp3)

    return k(idx2d, token_table)


def kernel(inputs, token_table, pos_table):
    idx2d = inputs.reshape(_ROWS // _GCH, _GCH).astype(jnp.int32)
    gathered = _sc_gather(idx2d, token_table)
    return gathered.reshape(_B, _L, _EMB) + pos_table[None, :, :]


# speedup vs baseline: 1.0970x; 1.0970x over previous
import functools

import jax
import jax.numpy as jnp
from jax import lax
from jax.experimental import pallas as pl
from jax.experimental.pallas import tpu as pltpu
from jax.experimental.pallas import tpu_sc as plsc

_B = 4
_L = 8192
_EMB = 64
_PAD = 128
_ROWS = _B * _L
_NW = 16
_CHUNK = _ROWS // _NW
_GCH = 128
_QROWS = 256
_NQ = _CHUNK // _QROWS


def _sc_gather(idx2d, tok_pad):
    mesh = plsc.VectorSubcoreMesh(
        core_axis_name="c", subcore_axis_name="s", num_cores=1
    )

    @functools.partial(
        pl.kernel,
        mesh=mesh,
        out_type=jax.ShapeDtypeStruct((_ROWS, _EMB), jnp.float32),
        scratch_types=[
            pltpu.VMEM((_CHUNK // _GCH, _GCH), jnp.int32),
            pltpu.VMEM((_QROWS, _PAD), jnp.float32),
            pltpu.VMEM((_QROWS, _PAD), jnp.float32),
            pltpu.SemaphoreType.DMA,
            pltpu.SemaphoreType.DMA,
        ],
        compiler_params=pltpu.CompilerParams(use_tc_tiling_on_sc=False),
    )
    def k(idx_hbm, tok_hbm, out_hbm, idx_v, rows_a, rows_b, sem_a, sem_b):
        w = lax.axis_index("s")
        pltpu.sync_copy(idx_hbm.at[pl.ds(w * (_CHUNK // _GCH), _CHUNK // _GCH)], idx_v)
        bufs = (rows_a, rows_b)
        sems = (sem_a, sem_b)

        def fire(q):
            buf, sem = bufs[q % 2], sems[q % 2]
            return [
                pltpu.async_copy(
                    tok_hbm.at[idx_v.at[q * 2 + h]],
                    buf.at[pl.ds(h * _GCH, _GCH)],
                    sem,
                )
                for h in range(_QROWS // _GCH)
            ]

        def drain(q, cps):
            for cp in cps:
                cp.wait()
            pltpu.sync_copy(
                bufs[q % 2].at[:, pl.ds(0, _EMB)],
                out_hbm.at[pl.ds(w * _CHUNK + q * _QROWS, _QROWS)],
            )

        pending = fire(0)
        for q in range(_NQ):
            nxt = fire(q + 1) if q + 1 < _NQ else None
            drain(q, pending)
            pending = nxt

    return k(idx2d, tok_pad)


def kernel(inputs, token_table, pos_table):
    tok_pad = jnp.pad(token_table, ((0, 0), (0, _PAD - _EMB)))
    idx2d = inputs.reshape(_ROWS // _GCH, _GCH).astype(jnp.int32)
    gathered = _sc_gather(idx2d, tok_pad)
    return gathered.reshape(_B, _L, _EMB) + pos_table[None, :, :]
